# Initial kernel scaffold; baseline (speedup 1.0000x reference)
import jax
import jax.numpy as jnp
from jax.experimental import pallas as pl


def kernel(ids, embed):
    raise NotImplementedError("write your pallas kernel here")


# speedup vs baseline: 6.3977x; 6.3977x over previous
import functools

import jax
import jax.numpy as jnp
from jax import lax
from jax.experimental import pallas as pl
from jax.experimental.pallas import tpu as pltpu
from jax.experimental.pallas import tpu_sc as plsc

NC, NS = 2, 16
NW = NC * NS
LANES = 128
K = 16


@functools.lru_cache(maxsize=None)
def _make_kernel(nblocks: int, embed_dim: int):
    mesh = plsc.VectorSubcoreMesh(core_axis_name="c", subcore_axis_name="s")
    blocks_per_w = nblocks // NW
    nch = blocks_per_w // K

    @functools.partial(
        pl.kernel,
        out_type=jax.ShapeDtypeStruct((nblocks, LANES, embed_dim), jnp.float32),
        mesh=mesh,
        compiler_params=pltpu.CompilerParams(use_tc_tiling_on_sc=False),
        scratch_types=[
            pltpu.VMEM((K, LANES), jnp.int32),
            pltpu.VMEM((K, LANES, embed_dim), jnp.float32),
            pltpu.VMEM_SHARED((8, embed_dim), jnp.float32),
            pltpu.SemaphoreType.DMA,
        ],
    )
    def kern(ids_hbm, table_hbm, out_hbm, idx_v, rows_v, table_sp, sem):
        cid = lax.axis_index("c")
        sid = lax.axis_index("s")
        wid = sid * NC + cid
        w_base = wid * blocks_per_w

        @pl.when(sid == 0)
        def _():
            pltpu.sync_copy(table_hbm, table_sp)

        plsc.subcore_barrier()

        @pl.loop(0, nch)
        def chunk_loop(g):
            base = w_base + g * K
            pltpu.sync_copy(ids_hbm.at[pl.ds(base, K)], idx_v)
            gathers = [
                pltpu.async_copy(table_sp.at[idx_v.at[k]], rows_v.at[k], sem)
                for k in range(K)
            ]
            for cp in gathers:
                cp.wait()
            pltpu.sync_copy(rows_v, out_hbm.at[pl.ds(base, K)])

    return kern


def kernel(ids, embed):
    n = ids.shape[0] * ids.shape[1]
    nblocks = n // LANES
    ids_b = ids.astype(jnp.int32).reshape(nblocks, LANES)
    out = _make_kernel(nblocks, embed.shape[1])(ids_b, embed)
    return out.reshape(ids.shape[0], ids.shape[1], embed.shape[1])
